# Initial kernel scaffold; baseline (speedup 1.0000x reference)
import jax
import jax.numpy as jnp
from jax.experimental import pallas as pl


def kernel(lattice, fracs, species, batch_indices, num_atoms_list, emb, Wm1, bm1, Wm2, bm2, Wu1, bu1, Wu2, bu2, ln_g, ln_b, Wf1, bf1, Wf2, bf2):
    raise NotImplementedError("write your pallas kernel here")


# speedup vs baseline: 10.9750x; 10.9750x over previous
import jax
import jax.numpy as jnp
from jax.experimental import pallas as pl
from jax.experimental.pallas import tpu as pltpu

_B = 64
_NPG = 200
_K = 12
_ND = 128
_ED = 64
_LAT = 128


def _silu(x):
    return x * jax.nn.sigmoid(x)


def _graph_kernel(lat_ref, fr_ref, frt_ref, sp_ref, emb_ref,
                  Wm1_ref, bm1_ref, Wm2_ref, bm2_ref,
                  Wu1_ref, bu1_ref, Wu2_ref, bu2_ref,
                  lng_ref, lnb_ref, out_ref):
    f32 = jnp.float32
    c0 = jnp.zeros((_NPG, _NPG), f32)
    c1 = jnp.zeros((_NPG, _NPG), f32)
    c2 = jnp.zeros((_NPG, _NPG), f32)
    for d in range(3):
        col = fr_ref[0, :, d:d + 1]
        row = frt_ref[0, d:d + 1, :]
        dd = col - row
        dd = dd - jnp.round(dd)
        c0 = c0 + dd * lat_ref[0, d, 0]
        c1 = c1 + dd * lat_ref[0, d, 1]
        c2 = c2 + dd * lat_ref[0, d, 2]
    dist = jnp.sqrt(c0 * c0 + c1 * c1 + c2 * c2 + 1e-12)

    rows = jax.lax.broadcasted_iota(jnp.int32, (_NPG, _NPG), 0)
    cols = jax.lax.broadcasted_iota(jnp.int32, (_NPG, _NPG), 1)
    inf = f32(jnp.inf)
    dist = jnp.where(rows == cols, inf, dist)

    oh_list = []
    dk_list = []
    d = dist
    for _ in range(_K):
        mn = jnp.min(d, axis=1, keepdims=True)
        hit = d == mn
        idx = jnp.min(jnp.where(hit, cols, jnp.int32(2 ** 30)),
                      axis=1, keepdims=True)
        sel = cols == idx
        d = jnp.where(sel, inf, d)
        oh_list.append(sel.astype(f32))
        dk_list.append(mn)
    onehot = jnp.concatenate(oh_list, axis=0)
    dks = jnp.concatenate(dk_list, axis=0)

    step = f32(8.0 / (_ED - 1))
    offs = jax.lax.broadcasted_iota(jnp.int32, (1, _ED), 1).astype(f32) * step
    coeff = f32(-0.5) / (step * step)
    ef = jnp.exp(coeff * (dks - offs) ** 2)

    spcol = sp_ref[0]
    emb_rows = emb_ref[...].shape[0]
    sp_oh = (spcol == jax.lax.broadcasted_iota(jnp.int32, (_NPG, emb_rows), 1)
             ).astype(f32)
    x = jnp.dot(sp_oh, emb_ref[...], preferred_element_type=f32)

    for l in range(3):
        Wm1 = Wm1_ref[l]
        ys = jnp.dot(x, Wm1[0:_ND], preferred_element_type=f32)
        yd = jnp.dot(x, Wm1[_ND:2 * _ND], preferred_element_type=f32)
        efp = jnp.dot(ef, Wm1[2 * _ND:], preferred_element_type=f32)
        g = jnp.dot(onehot, ys, preferred_element_type=f32)
        pre = g + efp + bm1_ref[l:l + 1, :]
        pre3 = pre.reshape(_K, _NPG, _ND) + yd[None, :, :]
        hsum = jnp.sum(_silu(pre3), axis=0)
        agg = (jnp.dot(hsum, Wm2_ref[l], preferred_element_type=f32)
               + f32(_K) * bm2_ref[l:l + 1, :])
        t = _silu(jnp.dot(agg, Wu1_ref[l], preferred_element_type=f32)
                  + bu1_ref[l:l + 1, :])
        upd = jnp.dot(t, Wu2_ref[l], preferred_element_type=f32) + bu2_ref[l:l + 1, :]
        z = x + upd
        mu = jnp.mean(z, axis=-1, keepdims=True)
        var = jnp.mean((z - mu) ** 2, axis=-1, keepdims=True)
        x = (z - mu) / jnp.sqrt(var + 1e-5) * lng_ref[l:l + 1, :] + lnb_ref[l:l + 1, :]

    out_ref[...] = jnp.mean(x, axis=0, keepdims=True).reshape(1, 1, _ND)


def _head_kernel(h_ref, w1_ref, b1_ref, w2_ref, b2_ref, o_ref):
    h = h_ref[...]
    t = _silu(jnp.dot(h, w1_ref[...], preferred_element_type=jnp.float32)
              + b1_ref[...])
    o_ref[...] = jnp.dot(t, w2_ref[...], preferred_element_type=jnp.float32) + b2_ref[...]


def kernel(lattice, fracs, species, batch_indices, num_atoms_list, emb,
           Wm1, bm1, Wm2, bm2, Wu1, bu1, Wu2, bu2, ln_g, ln_b,
           Wf1, bf1, Wf2, bf2):
    f32 = jnp.float32
    fr = fracs.reshape(_B, _NPG, 3).astype(f32)
    frt = fr.transpose(0, 2, 1)
    sp = species.astype(jnp.int32).reshape(_B, _NPG, 1)
    lat = lattice.astype(f32)

    full = lambda a: pl.BlockSpec(a.shape, lambda b: (0,) * a.ndim)
    pooled = pl.pallas_call(
        _graph_kernel,
        grid=(_B,),
        in_specs=[
            pl.BlockSpec((1, 3, 3), lambda b: (b, 0, 0)),
            pl.BlockSpec((1, _NPG, 3), lambda b: (b, 0, 0)),
            pl.BlockSpec((1, 3, _NPG), lambda b: (b, 0, 0)),
            pl.BlockSpec((1, _NPG, 1), lambda b: (b, 0, 0)),
            full(emb), full(Wm1), full(bm1), full(Wm2), full(bm2),
            full(Wu1), full(bu1), full(Wu2), full(bu2),
            full(ln_g), full(ln_b),
        ],
        out_specs=pl.BlockSpec((1, 1, _ND), lambda b: (b, 0, 0)),
        out_shape=jax.ShapeDtypeStruct((_B, 1, _ND), f32),
        compiler_params=pltpu.CompilerParams(
            dimension_semantics=("parallel",)),
    )(lat, fr, frt, sp, emb.astype(f32),
      Wm1.astype(f32), bm1.astype(f32), Wm2.astype(f32), bm2.astype(f32),
      Wu1.astype(f32), bu1.astype(f32), Wu2.astype(f32), bu2.astype(f32),
      ln_g.astype(f32), ln_b.astype(f32))

    h = jnp.concatenate([pooled.reshape(_B, _ND), lat.reshape(_B, 9)], axis=1)
    out = pl.pallas_call(
        _head_kernel,
        in_specs=[pl.BlockSpec(h.shape, lambda: (0, 0)),
                  pl.BlockSpec(Wf1.shape, lambda: (0, 0)),
                  pl.BlockSpec((1, Wf1.shape[1]), lambda: (0, 0)),
                  pl.BlockSpec(Wf2.shape, lambda: (0, 0)),
                  pl.BlockSpec((1, Wf2.shape[1]), lambda: (0, 0))],
        out_specs=pl.BlockSpec((_B, Wf2.shape[1]), lambda: (0, 0)),
        out_shape=jax.ShapeDtypeStruct((_B, Wf2.shape[1]), f32),
    )(h, Wf1.astype(f32), bf1.astype(f32).reshape(1, -1),
      Wf2.astype(f32), bf2.astype(f32).reshape(1, -1))

    return (out[:, :_LAT], out[:, _LAT:])


# speedup vs baseline: 15.1703x; 1.3823x over previous
import jax
import jax.numpy as jnp
from jax.experimental import pallas as pl
from jax.experimental.pallas import tpu as pltpu

_B = 64
_NPG = 200
_K = 12
_ND = 128
_ED = 64
_LAT = 128
_GPP = 2


def _silu(x):
    return x * (0.5 * jnp.tanh(0.5 * x) + 0.5)


def _one_graph(g, lat_ref, fr_ref, frt_ref, sp_ref, emb_ref,
               Wm1_ref, bm1_ref, Wm2_ref, bm2_ref,
               Wu1_ref, bu1_ref, Wu2_ref, bu2_ref,
               lng_ref, lnb_ref):
    f32 = jnp.float32
    c0 = jnp.zeros((_NPG, _NPG), f32)
    c1 = jnp.zeros((_NPG, _NPG), f32)
    c2 = jnp.zeros((_NPG, _NPG), f32)
    for d in range(3):
        col = fr_ref[g, :, d:d + 1]
        row = frt_ref[g, d:d + 1, :]
        dd = col - row
        dd = dd - jnp.round(dd)
        c0 = c0 + dd * lat_ref[g, d, 0]
        c1 = c1 + dd * lat_ref[g, d, 1]
        c2 = c2 + dd * lat_ref[g, d, 2]
    d2 = c0 * c0 + c1 * c1 + c2 * c2

    rows = jax.lax.broadcasted_iota(jnp.int32, (_NPG, _NPG), 0)
    cols = jax.lax.broadcasted_iota(jnp.int32, (_NPG, _NPG), 1)
    inf = f32(jnp.inf)
    d2 = jnp.where(rows == cols, inf, d2)

    oh_list = []
    dk_list = []
    for _ in range(_K):
        mn = jnp.min(d2, axis=1, keepdims=True)
        sel = d2 == mn
        d2 = jnp.where(sel, inf, d2)
        oh_list.append(sel.astype(f32))
        dk_list.append(mn)
    onehot = jnp.concatenate(oh_list, axis=0)
    dks = jnp.sqrt(jnp.concatenate(dk_list, axis=0) + 1e-12)

    step = f32(8.0 / (_ED - 1))
    offs = jax.lax.broadcasted_iota(jnp.int32, (1, _ED), 1).astype(f32) * step
    coeff = f32(-0.5) / (step * step)
    ef = jnp.exp(coeff * (dks - offs) ** 2)

    spcol = sp_ref[g]
    emb_rows = emb_ref[...].shape[0]
    sp_oh = (spcol == jax.lax.broadcasted_iota(jnp.int32, (_NPG, emb_rows), 1)
             ).astype(f32)
    x = jnp.dot(sp_oh, emb_ref[...], preferred_element_type=f32)

    Wef_all = jnp.concatenate([Wm1_ref[l][2 * _ND:] for l in range(3)], axis=1)
    efp_all = jnp.dot(ef, Wef_all, preferred_element_type=f32)

    for l in range(3):
        Wm1 = Wm1_ref[l]
        ys = jnp.dot(x, Wm1[0:_ND], preferred_element_type=f32)
        yd = jnp.dot(x, Wm1[_ND:2 * _ND], preferred_element_type=f32)
        efp = efp_all[:, l * _ND:(l + 1) * _ND]
        gth = jnp.dot(onehot, ys, preferred_element_type=f32)
        pre = gth + efp + bm1_ref[l:l + 1, :]
        pre3 = pre.reshape(_K, _NPG, _ND) + yd[None, :, :]
        hsum = jnp.sum(_silu(pre3), axis=0)
        agg = (jnp.dot(hsum, Wm2_ref[l], preferred_element_type=f32)
               + f32(_K) * bm2_ref[l:l + 1, :])
        t = _silu(jnp.dot(agg, Wu1_ref[l], preferred_element_type=f32)
                  + bu1_ref[l:l + 1, :])
        upd = jnp.dot(t, Wu2_ref[l], preferred_element_type=f32) + bu2_ref[l:l + 1, :]
        z = x + upd
        mu = jnp.mean(z, axis=-1, keepdims=True)
        var = jnp.mean((z - mu) ** 2, axis=-1, keepdims=True)
        x = (z - mu) / jnp.sqrt(var + 1e-5) * lng_ref[l:l + 1, :] + lnb_ref[l:l + 1, :]

    return jnp.mean(x, axis=0, keepdims=True).reshape(1, 1, _ND)


def _graph_kernel(lat_ref, fr_ref, frt_ref, sp_ref, emb_ref,
                  Wm1_ref, bm1_ref, Wm2_ref, bm2_ref,
                  Wu1_ref, bu1_ref, Wu2_ref, bu2_ref,
                  lng_ref, lnb_ref, out_ref):
    pooled = [
        _one_graph(g, lat_ref, fr_ref, frt_ref, sp_ref, emb_ref,
                   Wm1_ref, bm1_ref, Wm2_ref, bm2_ref,
                   Wu1_ref, bu1_ref, Wu2_ref, bu2_ref,
                   lng_ref, lnb_ref)
        for g in range(_GPP)
    ]
    out_ref[...] = jnp.concatenate(pooled, axis=0)


def _head_kernel(h_ref, w1_ref, b1_ref, w2_ref, b2_ref, o_ref):
    h = h_ref[...]
    t = _silu(jnp.dot(h, w1_ref[...], preferred_element_type=jnp.float32)
              + b1_ref[...])
    o_ref[...] = jnp.dot(t, w2_ref[...], preferred_element_type=jnp.float32) + b2_ref[...]


def kernel(lattice, fracs, species, batch_indices, num_atoms_list, emb,
           Wm1, bm1, Wm2, bm2, Wu1, bu1, Wu2, bu2, ln_g, ln_b,
           Wf1, bf1, Wf2, bf2):
    f32 = jnp.float32
    fr = fracs.reshape(_B, _NPG, 3).astype(f32)
    frt = fr.transpose(0, 2, 1)
    sp = species.astype(jnp.int32).reshape(_B, _NPG, 1)
    lat = lattice.astype(f32)

    full = lambda a: pl.BlockSpec(a.shape, lambda b: (0,) * a.ndim)
    pooled = pl.pallas_call(
        _graph_kernel,
        grid=(_B // _GPP,),
        in_specs=[
            pl.BlockSpec((_GPP, 3, 3), lambda b: (b, 0, 0)),
            pl.BlockSpec((_GPP, _NPG, 3), lambda b: (b, 0, 0)),
            pl.BlockSpec((_GPP, 3, _NPG), lambda b: (b, 0, 0)),
            pl.BlockSpec((_GPP, _NPG, 1), lambda b: (b, 0, 0)),
            full(emb), full(Wm1), full(bm1), full(Wm2), full(bm2),
            full(Wu1), full(bu1), full(Wu2), full(bu2),
            full(ln_g), full(ln_b),
        ],
        out_specs=pl.BlockSpec((_GPP, 1, _ND), lambda b: (b, 0, 0)),
        out_shape=jax.ShapeDtypeStruct((_B, 1, _ND), f32),
        compiler_params=pltpu.CompilerParams(
            dimension_semantics=("parallel",)),
    )(lat, fr, frt, sp, emb.astype(f32),
      Wm1.astype(f32), bm1.astype(f32), Wm2.astype(f32), bm2.astype(f32),
      Wu1.astype(f32), bu1.astype(f32), Wu2.astype(f32), bu2.astype(f32),
      ln_g.astype(f32), ln_b.astype(f32))

    h = jnp.concatenate([pooled.reshape(_B, _ND), lat.reshape(_B, 9)], axis=1)
    out = pl.pallas_call(
        _head_kernel,
        in_specs=[pl.BlockSpec(h.shape, lambda: (0, 0)),
                  pl.BlockSpec(Wf1.shape, lambda: (0, 0)),
                  pl.BlockSpec((1, Wf1.shape[1]), lambda: (0, 0)),
                  pl.BlockSpec(Wf2.shape, lambda: (0, 0)),
                  pl.BlockSpec((1, Wf2.shape[1]), lambda: (0, 0))],
        out_specs=pl.BlockSpec((_B, Wf2.shape[1]), lambda: (0, 0)),
        out_shape=jax.ShapeDtypeStruct((_B, Wf2.shape[1]), f32),
    )(h, Wf1.astype(f32), bf1.astype(f32).reshape(1, -1),
      Wf2.astype(f32), bf2.astype(f32).reshape(1, -1))

    return (out[:, :_LAT], out[:, _LAT:])


# speedup vs baseline: 17.0137x; 1.1215x over previous
import jax
import jax.numpy as jnp
from jax.experimental import pallas as pl
from jax.experimental.pallas import tpu as pltpu

_B = 64
_NPG = 200
_K = 12
_ND = 128
_ED = 64
_LAT = 128
_GPP = 4


def _silu(x):
    return x * (0.5 * jnp.tanh(0.5 * x) + 0.5)


def _one_graph(g, lat_ref, fr_ref, frt_ref, sp_ref, emb_ref,
               Wm1_ref, bm1_ref, Wm2_ref, bm2_ref,
               Wu1_ref, bu1_ref, Wu2_ref, bu2_ref,
               lng_ref, lnb_ref):
    f32 = jnp.float32
    c0 = jnp.zeros((_NPG, _NPG), f32)
    c1 = jnp.zeros((_NPG, _NPG), f32)
    c2 = jnp.zeros((_NPG, _NPG), f32)
    for d in range(3):
        col = fr_ref[g, :, d:d + 1]
        row = frt_ref[g, d:d + 1, :]
        dd = col - row
        dd = dd - jnp.round(dd)
        c0 = c0 + dd * lat_ref[g, d, 0]
        c1 = c1 + dd * lat_ref[g, d, 1]
        c2 = c2 + dd * lat_ref[g, d, 2]
    d2 = c0 * c0 + c1 * c1 + c2 * c2

    rows = jax.lax.broadcasted_iota(jnp.int32, (_NPG, _NPG), 0)
    cols = jax.lax.broadcasted_iota(jnp.int32, (_NPG, _NPG), 1)
    inf = f32(jnp.inf)
    d2 = jnp.where(rows == cols, inf, d2)

    oh_list = []
    dk_list = []
    for _ in range(_K):
        mn = jnp.min(d2, axis=1, keepdims=True)
        sel = d2 == mn
        d2 = jnp.where(sel, inf, d2)
        oh_list.append(sel.astype(f32))
        dk_list.append(mn)
    dks = jnp.sqrt(jnp.concatenate(dk_list, axis=0) + 1e-12)

    step = f32(8.0 / (_ED - 1))
    offs = jax.lax.broadcasted_iota(jnp.int32, (1, _ED), 1).astype(f32) * step
    coeff = f32(-0.5) / (step * step)
    ef = jnp.exp(coeff * (dks - offs) ** 2)

    spcol = sp_ref[g]
    emb_rows = emb_ref[...].shape[0]
    sp_oh = (spcol == jax.lax.broadcasted_iota(jnp.int32, (_NPG, emb_rows), 1)
             ).astype(f32)
    x = jnp.dot(sp_oh, emb_ref[...], preferred_element_type=f32)

    Wef_all = jnp.concatenate([Wm1_ref[l][2 * _ND:] for l in range(3)], axis=1)
    efp_all = jnp.dot(ef, Wef_all, preferred_element_type=f32)

    for l in range(3):
        Wm1 = Wm1_ref[l]
        ys = jnp.dot(x, Wm1[0:_ND], preferred_element_type=f32)
        yd = jnp.dot(x, Wm1[_ND:2 * _ND], preferred_element_type=f32)
        base = yd + bm1_ref[l:l + 1, :]
        hsum = jnp.zeros((_NPG, _ND), f32)
        for k in range(_K):
            gk = jnp.dot(oh_list[k], ys, preferred_element_type=f32)
            efpk = efp_all[k * _NPG:(k + 1) * _NPG, l * _ND:(l + 1) * _ND]
            hsum = hsum + _silu(gk + efpk + base)
        agg = (jnp.dot(hsum, Wm2_ref[l], preferred_element_type=f32)
               + f32(_K) * bm2_ref[l:l + 1, :])
        t = _silu(jnp.dot(agg, Wu1_ref[l], preferred_element_type=f32)
                  + bu1_ref[l:l + 1, :])
        upd = jnp.dot(t, Wu2_ref[l], preferred_element_type=f32) + bu2_ref[l:l + 1, :]
        z = x + upd
        mu = jnp.mean(z, axis=-1, keepdims=True)
        var = jnp.mean((z - mu) ** 2, axis=-1, keepdims=True)
        x = (z - mu) / jnp.sqrt(var + 1e-5) * lng_ref[l:l + 1, :] + lnb_ref[l:l + 1, :]

    return jnp.mean(x, axis=0, keepdims=True).reshape(1, 1, _ND)


def _graph_kernel(lat_ref, fr_ref, frt_ref, sp_ref, emb_ref,
                  Wm1_ref, bm1_ref, Wm2_ref, bm2_ref,
                  Wu1_ref, bu1_ref, Wu2_ref, bu2_ref,
                  lng_ref, lnb_ref, out_ref):
    pooled = [
        _one_graph(g, lat_ref, fr_ref, frt_ref, sp_ref, emb_ref,
                   Wm1_ref, bm1_ref, Wm2_ref, bm2_ref,
                   Wu1_ref, bu1_ref, Wu2_ref, bu2_ref,
                   lng_ref, lnb_ref)
        for g in range(_GPP)
    ]
    out_ref[...] = jnp.concatenate(pooled, axis=0)


def _head_kernel(h_ref, w1_ref, b1_ref, w2_ref, b2_ref, o_ref):
    h = h_ref[...]
    t = _silu(jnp.dot(h, w1_ref[...], preferred_element_type=jnp.float32)
              + b1_ref[...])
    o_ref[...] = jnp.dot(t, w2_ref[...], preferred_element_type=jnp.float32) + b2_ref[...]


def kernel(lattice, fracs, species, batch_indices, num_atoms_list, emb,
           Wm1, bm1, Wm2, bm2, Wu1, bu1, Wu2, bu2, ln_g, ln_b,
           Wf1, bf1, Wf2, bf2):
    f32 = jnp.float32
    fr = fracs.reshape(_B, _NPG, 3).astype(f32)
    frt = fr.transpose(0, 2, 1)
    sp = species.astype(jnp.int32).reshape(_B, _NPG, 1)
    lat = lattice.astype(f32)

    full = lambda a: pl.BlockSpec(a.shape, lambda b: (0,) * a.ndim)
    pooled = pl.pallas_call(
        _graph_kernel,
        grid=(_B // _GPP,),
        in_specs=[
            pl.BlockSpec((_GPP, 3, 3), lambda b: (b, 0, 0)),
            pl.BlockSpec((_GPP, _NPG, 3), lambda b: (b, 0, 0)),
            pl.BlockSpec((_GPP, 3, _NPG), lambda b: (b, 0, 0)),
            pl.BlockSpec((_GPP, _NPG, 1), lambda b: (b, 0, 0)),
            full(emb), full(Wm1), full(bm1), full(Wm2), full(bm2),
            full(Wu1), full(bu1), full(Wu2), full(bu2),
            full(ln_g), full(ln_b),
        ],
        out_specs=pl.BlockSpec((_GPP, 1, _ND), lambda b: (b, 0, 0)),
        out_shape=jax.ShapeDtypeStruct((_B, 1, _ND), f32),
        compiler_params=pltpu.CompilerParams(
            dimension_semantics=("parallel",)),
    )(lat, fr, frt, sp, emb.astype(f32),
      Wm1.astype(f32), bm1.astype(f32), Wm2.astype(f32), bm2.astype(f32),
      Wu1.astype(f32), bu1.astype(f32), Wu2.astype(f32), bu2.astype(f32),
      ln_g.astype(f32), ln_b.astype(f32))

    h = jnp.concatenate([pooled.reshape(_B, _ND), lat.reshape(_B, 9)], axis=1)
    out = pl.pallas_call(
        _head_kernel,
        in_specs=[pl.BlockSpec(h.shape, lambda: (0, 0)),
                  pl.BlockSpec(Wf1.shape, lambda: (0, 0)),
                  pl.BlockSpec((1, Wf1.shape[1]), lambda: (0, 0)),
                  pl.BlockSpec(Wf2.shape, lambda: (0, 0)),
                  pl.BlockSpec((1, Wf2.shape[1]), lambda: (0, 0))],
        out_specs=pl.BlockSpec((_B, Wf2.shape[1]), lambda: (0, 0)),
        out_shape=jax.ShapeDtypeStruct((_B, Wf2.shape[1]), f32),
    )(h, Wf1.astype(f32), bf1.astype(f32).reshape(1, -1),
      Wf2.astype(f32), bf2.astype(f32).reshape(1, -1))

    return (out[:, :_LAT], out[:, _LAT:])


# speedup vs baseline: 17.4170x; 1.0237x over previous
import jax
import jax.numpy as jnp
from jax.experimental import pallas as pl
from jax.experimental.pallas import tpu as pltpu

_B = 64
_NPG = 200
_K = 12
_ND = 128
_ED = 64
_LAT = 128
_GPP = 8


def _silu(x):
    return x * (0.5 * jnp.tanh(0.5 * x) + 0.5)


def _one_graph(g, lat_ref, fr_ref, frt_ref, sp_ref, emb_ref,
               Wm1_ref, bm1_ref, Wm2_ref, bm2_ref,
               Wu1_ref, bu1_ref, Wu2_ref, bu2_ref,
               lng_ref, lnb_ref):
    f32 = jnp.float32
    c0 = jnp.zeros((_NPG, _NPG), f32)
    c1 = jnp.zeros((_NPG, _NPG), f32)
    c2 = jnp.zeros((_NPG, _NPG), f32)
    for d in range(3):
        col = fr_ref[g, :, d:d + 1]
        row = frt_ref[g, d:d + 1, :]
        dd = col - row
        dd = dd - jnp.round(dd)
        c0 = c0 + dd * lat_ref[g, d, 0]
        c1 = c1 + dd * lat_ref[g, d, 1]
        c2 = c2 + dd * lat_ref[g, d, 2]
    d2 = c0 * c0 + c1 * c1 + c2 * c2

    rows = jax.lax.broadcasted_iota(jnp.int32, (_NPG, _NPG), 0)
    cols = jax.lax.broadcasted_iota(jnp.int32, (_NPG, _NPG), 1)
    inf = f32(jnp.inf)
    d2 = jnp.where(rows == cols, inf, d2)

    oh_list = []
    dk_list = []
    for _ in range(_K):
        mn = jnp.min(d2, axis=1, keepdims=True)
        sel = d2 == mn
        d2 = jnp.where(sel, inf, d2)
        oh_list.append(sel.astype(jnp.bfloat16))
        dk_list.append(mn)
    dks = jnp.sqrt(jnp.concatenate(dk_list, axis=0) + 1e-12)

    step = f32(8.0 / (_ED - 1))
    offs = jax.lax.broadcasted_iota(jnp.int32, (1, _ED), 1).astype(f32) * step
    coeff = f32(-0.5) / (step * step)
    ef = jnp.exp(coeff * (dks - offs) ** 2)

    spcol = sp_ref[g]
    emb_rows = emb_ref[...].shape[0]
    sp_oh = (spcol == jax.lax.broadcasted_iota(jnp.int32, (_NPG, emb_rows), 1)
             ).astype(f32)
    x = jnp.dot(sp_oh, emb_ref[...], preferred_element_type=f32)

    Wef_all = jnp.concatenate([Wm1_ref[l][2 * _ND:] for l in range(3)], axis=1)
    efp_all = jnp.dot(ef, Wef_all, preferred_element_type=f32)

    for l in range(3):
        Wm1 = Wm1_ref[l]
        ys = jnp.dot(x, Wm1[0:_ND], preferred_element_type=f32)
        yd = jnp.dot(x, Wm1[_ND:2 * _ND], preferred_element_type=f32)
        base = yd + bm1_ref[l:l + 1, :]
        ys_bf = ys.astype(jnp.bfloat16)
        hsum = jnp.zeros((_NPG, _ND), f32)
        for k in range(_K):
            gk = jnp.dot(oh_list[k], ys_bf, preferred_element_type=f32)
            efpk = efp_all[k * _NPG:(k + 1) * _NPG, l * _ND:(l + 1) * _ND]
            hsum = hsum + _silu(gk + efpk + base)
        agg = (jnp.dot(hsum, Wm2_ref[l], preferred_element_type=f32)
               + f32(_K) * bm2_ref[l:l + 1, :])
        t = _silu(jnp.dot(agg, Wu1_ref[l], preferred_element_type=f32)
                  + bu1_ref[l:l + 1, :])
        upd = jnp.dot(t, Wu2_ref[l], preferred_element_type=f32) + bu2_ref[l:l + 1, :]
        z = x + upd
        mu = jnp.mean(z, axis=-1, keepdims=True)
        var = jnp.mean((z - mu) ** 2, axis=-1, keepdims=True)
        x = (z - mu) / jnp.sqrt(var + 1e-5) * lng_ref[l:l + 1, :] + lnb_ref[l:l + 1, :]

    return jnp.mean(x, axis=0, keepdims=True).reshape(1, 1, _ND)


def _graph_kernel(lat_ref, fr_ref, frt_ref, sp_ref, emb_ref,
                  Wm1_ref, bm1_ref, Wm2_ref, bm2_ref,
                  Wu1_ref, bu1_ref, Wu2_ref, bu2_ref,
                  lng_ref, lnb_ref, out_ref):
    pooled = [
        _one_graph(g, lat_ref, fr_ref, frt_ref, sp_ref, emb_ref,
                   Wm1_ref, bm1_ref, Wm2_ref, bm2_ref,
                   Wu1_ref, bu1_ref, Wu2_ref, bu2_ref,
                   lng_ref, lnb_ref)
        for g in range(_GPP)
    ]
    out_ref[...] = jnp.concatenate(pooled, axis=0)


def _head_kernel(h_ref, w1_ref, b1_ref, w2_ref, b2_ref, o_ref):
    h = h_ref[...]
    t = _silu(jnp.dot(h, w1_ref[...], preferred_element_type=jnp.float32)
              + b1_ref[...])
    o_ref[...] = jnp.dot(t, w2_ref[...], preferred_element_type=jnp.float32) + b2_ref[...]


def kernel(lattice, fracs, species, batch_indices, num_atoms_list, emb,
           Wm1, bm1, Wm2, bm2, Wu1, bu1, Wu2, bu2, ln_g, ln_b,
           Wf1, bf1, Wf2, bf2):
    f32 = jnp.float32
    fr = fracs.reshape(_B, _NPG, 3).astype(f32)
    frt = fr.transpose(0, 2, 1)
    sp = species.astype(jnp.int32).reshape(_B, _NPG, 1)
    lat = lattice.astype(f32)

    full = lambda a: pl.BlockSpec(a.shape, lambda b: (0,) * a.ndim)
    pooled = pl.pallas_call(
        _graph_kernel,
        grid=(_B // _GPP,),
        in_specs=[
            pl.BlockSpec((_GPP, 3, 3), lambda b: (b, 0, 0)),
            pl.BlockSpec((_GPP, _NPG, 3), lambda b: (b, 0, 0)),
            pl.BlockSpec((_GPP, 3, _NPG), lambda b: (b, 0, 0)),
            pl.BlockSpec((_GPP, _NPG, 1), lambda b: (b, 0, 0)),
            full(emb), full(Wm1), full(bm1), full(Wm2), full(bm2),
            full(Wu1), full(bu1), full(Wu2), full(bu2),
            full(ln_g), full(ln_b),
        ],
        out_specs=pl.BlockSpec((_GPP, 1, _ND), lambda b: (b, 0, 0)),
        out_shape=jax.ShapeDtypeStruct((_B, 1, _ND), f32),
        compiler_params=pltpu.CompilerParams(
            dimension_semantics=("parallel",)),
    )(lat, fr, frt, sp, emb.astype(f32),
      Wm1.astype(f32), bm1.astype(f32), Wm2.astype(f32), bm2.astype(f32),
      Wu1.astype(f32), bu1.astype(f32), Wu2.astype(f32), bu2.astype(f32),
      ln_g.astype(f32), ln_b.astype(f32))

    h = jnp.concatenate([pooled.reshape(_B, _ND), lat.reshape(_B, 9)], axis=1)
    out = pl.pallas_call(
        _head_kernel,
        in_specs=[pl.BlockSpec(h.shape, lambda: (0, 0)),
                  pl.BlockSpec(Wf1.shape, lambda: (0, 0)),
                  pl.BlockSpec((1, Wf1.shape[1]), lambda: (0, 0)),
                  pl.BlockSpec(Wf2.shape, lambda: (0, 0)),
                  pl.BlockSpec((1, Wf2.shape[1]), lambda: (0, 0))],
        out_specs=pl.BlockSpec((_B, Wf2.shape[1]), lambda: (0, 0)),
        out_shape=jax.ShapeDtypeStruct((_B, Wf2.shape[1]), f32),
    )(h, Wf1.astype(f32), bf1.astype(f32).reshape(1, -1),
      Wf2.astype(f32), bf2.astype(f32).reshape(1, -1))

    return (out[:, :_LAT], out[:, _LAT:])


# speedup vs baseline: 17.9455x; 1.0303x over previous
import jax
import jax.numpy as jnp
from jax.experimental import pallas as pl
from jax.experimental.pallas import tpu as pltpu

_B = 64
_NPG = 200
_K = 12
_ND = 128
_ED = 64
_LAT = 128
_GPP = 8


def _silu(x):
    return x * (0.5 * jnp.tanh(0.5 * x) + 0.5)


def _one_graph(g, lat_ref, fr_ref, frt_ref, sp_ref, emb_ref,
               Wm1_ref, bm1_ref, Wm2_ref, bm2_ref,
               Wu1_ref, bu1_ref, Wu2_ref, bu2_ref,
               lng_ref, lnb_ref):
    f32 = jnp.float32
    c0 = jnp.zeros((_NPG, _NPG), f32)
    c1 = jnp.zeros((_NPG, _NPG), f32)
    c2 = jnp.zeros((_NPG, _NPG), f32)
    for d in range(3):
        col = fr_ref[g, :, d:d + 1]
        row = frt_ref[g, d:d + 1, :]
        dd = col - row
        dd = dd - jnp.round(dd)
        c0 = c0 + dd * lat_ref[g, d, 0]
        c1 = c1 + dd * lat_ref[g, d, 1]
        c2 = c2 + dd * lat_ref[g, d, 2]
    d2 = c0 * c0 + c1 * c1 + c2 * c2

    rows = jax.lax.broadcasted_iota(jnp.int32, (_NPG, _NPG), 0)
    cols = jax.lax.broadcasted_iota(jnp.int32, (_NPG, _NPG), 1)
    inf = f32(jnp.inf)
    d2 = jnp.where(rows == cols, inf, d2)

    oh_list = []
    dk_list = []
    for _ in range(_K):
        mn = jnp.min(d2, axis=1, keepdims=True)
        sel = d2 == mn
        d2 = jnp.where(sel, inf, d2)
        oh_list.append(sel.astype(jnp.bfloat16))
        dk_list.append(mn)
    step = f32(8.0 / (_ED - 1))
    offs = jax.lax.broadcasted_iota(jnp.int32, (1, _ED), 1).astype(f32) * step
    coeff = f32(-0.5) / (step * step)
    ef = jnp.concatenate(
        [jnp.exp(coeff * (jnp.sqrt(mnk + 1e-12) - offs) ** 2)
         for mnk in dk_list], axis=0)

    spcol = sp_ref[g]
    emb_rows = emb_ref[...].shape[0]
    sp_oh = (spcol == jax.lax.broadcasted_iota(jnp.int32, (_NPG, emb_rows), 1)
             ).astype(f32)
    x = jnp.dot(sp_oh, emb_ref[...], preferred_element_type=f32)

    Wef_all = jnp.concatenate([Wm1_ref[l][2 * _ND:] for l in range(3)], axis=1)
    efp_all = jnp.dot(ef, Wef_all, preferred_element_type=f32)

    for l in range(3):
        Wm1 = Wm1_ref[l]
        ys = jnp.dot(x, Wm1[0:_ND], preferred_element_type=f32)
        yd = jnp.dot(x, Wm1[_ND:2 * _ND], preferred_element_type=f32)
        base = yd + bm1_ref[l:l + 1, :]
        ys_bf = ys.astype(jnp.bfloat16)
        hsum = jnp.zeros((_NPG, _ND), f32)
        for k in range(_K):
            gk = jnp.dot(oh_list[k], ys_bf, preferred_element_type=f32)
            efpk = efp_all[k * _NPG:(k + 1) * _NPG, l * _ND:(l + 1) * _ND]
            hsum = hsum + _silu(gk + efpk + base)
        agg = (jnp.dot(hsum, Wm2_ref[l], preferred_element_type=f32)
               + f32(_K) * bm2_ref[l:l + 1, :])
        t = _silu(jnp.dot(agg, Wu1_ref[l], preferred_element_type=f32)
                  + bu1_ref[l:l + 1, :])
        upd = jnp.dot(t, Wu2_ref[l], preferred_element_type=f32) + bu2_ref[l:l + 1, :]
        z = x + upd
        mu = jnp.mean(z, axis=-1, keepdims=True)
        var = jnp.mean((z - mu) ** 2, axis=-1, keepdims=True)
        x = (z - mu) / jnp.sqrt(var + 1e-5) * lng_ref[l:l + 1, :] + lnb_ref[l:l + 1, :]

    return jnp.mean(x, axis=0, keepdims=True).reshape(1, 1, _ND)


def _graph_kernel(lat_ref, fr_ref, frt_ref, sp_ref, emb_ref,
                  Wm1_ref, bm1_ref, Wm2_ref, bm2_ref,
                  Wu1_ref, bu1_ref, Wu2_ref, bu2_ref,
                  lng_ref, lnb_ref, out_ref):
    pooled = [
        _one_graph(g, lat_ref, fr_ref, frt_ref, sp_ref, emb_ref,
                   Wm1_ref, bm1_ref, Wm2_ref, bm2_ref,
                   Wu1_ref, bu1_ref, Wu2_ref, bu2_ref,
                   lng_ref, lnb_ref)
        for g in range(_GPP)
    ]
    out_ref[...] = jnp.concatenate(pooled, axis=0)


def _head_kernel(h_ref, w1_ref, b1_ref, w2_ref, b2_ref, o_ref):
    h = h_ref[...]
    t = _silu(jnp.dot(h, w1_ref[...], preferred_element_type=jnp.float32)
              + b1_ref[...])
    o_ref[...] = jnp.dot(t, w2_ref[...], preferred_element_type=jnp.float32) + b2_ref[...]


def kernel(lattice, fracs, species, batch_indices, num_atoms_list, emb,
           Wm1, bm1, Wm2, bm2, Wu1, bu1, Wu2, bu2, ln_g, ln_b,
           Wf1, bf1, Wf2, bf2):
    f32 = jnp.float32
    fr = fracs.reshape(_B, _NPG, 3).astype(f32)
    frt = fr.transpose(0, 2, 1)
    sp = species.astype(jnp.int32).reshape(_B, _NPG, 1)
    lat = lattice.astype(f32)

    full = lambda a: pl.BlockSpec(a.shape, lambda b: (0,) * a.ndim)
    pooled = pl.pallas_call(
        _graph_kernel,
        grid=(_B // _GPP,),
        in_specs=[
            pl.BlockSpec((_GPP, 3, 3), lambda b: (b, 0, 0)),
            pl.BlockSpec((_GPP, _NPG, 3), lambda b: (b, 0, 0)),
            pl.BlockSpec((_GPP, 3, _NPG), lambda b: (b, 0, 0)),
            pl.BlockSpec((_GPP, _NPG, 1), lambda b: (b, 0, 0)),
            full(emb), full(Wm1), full(bm1), full(Wm2), full(bm2),
            full(Wu1), full(bu1), full(Wu2), full(bu2),
            full(ln_g), full(ln_b),
        ],
        out_specs=pl.BlockSpec((_GPP, 1, _ND), lambda b: (b, 0, 0)),
        out_shape=jax.ShapeDtypeStruct((_B, 1, _ND), f32),
        compiler_params=pltpu.CompilerParams(
            dimension_semantics=("parallel",)),
    )(lat, fr, frt, sp, emb.astype(f32),
      Wm1.astype(f32), bm1.astype(f32), Wm2.astype(f32), bm2.astype(f32),
      Wu1.astype(f32), bu1.astype(f32), Wu2.astype(f32), bu2.astype(f32),
      ln_g.astype(f32), ln_b.astype(f32))

    h = jnp.concatenate([pooled.reshape(_B, _ND), lat.reshape(_B, 9)], axis=1)
    out = pl.pallas_call(
        _head_kernel,
        in_specs=[pl.BlockSpec(h.shape, lambda: (0, 0)),
                  pl.BlockSpec(Wf1.shape, lambda: (0, 0)),
                  pl.BlockSpec((1, Wf1.shape[1]), lambda: (0, 0)),
                  pl.BlockSpec(Wf2.shape, lambda: (0, 0)),
                  pl.BlockSpec((1, Wf2.shape[1]), lambda: (0, 0))],
        out_specs=pl.BlockSpec((_B, Wf2.shape[1]), lambda: (0, 0)),
        out_shape=jax.ShapeDtypeStruct((_B, Wf2.shape[1]), f32),
    )(h, Wf1.astype(f32), bf1.astype(f32).reshape(1, -1),
      Wf2.astype(f32), bf2.astype(f32).reshape(1, -1))

    return (out[:, :_LAT], out[:, _LAT:])
